# baseline (speedup 1.0000x reference)
import functools

import jax
import jax.numpy as jnp
from jax import lax
from jax.experimental import pallas as pl
from jax.experimental.pallas import tpu as pltpu
from jax.experimental.pallas import tpu_sc as plsc

T, D, FF, E, K = 2048, 1024, 512, 64, 2
SOFTCAP = 30.0
A = T * K
TM = 128
XS_ROWS = 4736

NC, NS = 2, 16
NW = NC * NS
TPW = T // NW
CPW = TPW // 2

def _sc_mesh():
    return plsc.VectorSubcoreMesh(core_axis_name="c", subcore_axis_name="s",
                                  num_cores=NC, num_subcores=NS)


def _dispatch_body(x_hbm, p1_hbm, p2_hbm, m1_hbm, m2_hbm, xs_hbm, ws_hbm,
                   idx1_v, idx2_v, mv1_v, mv2_v, rows_v, sem1, sem2):
    wid = lax.axis_index("s") * NC + lax.axis_index("c")
    base = wid * TPW
    pltpu.sync_copy(p1_hbm.at[pl.ds(base, TPW)], idx1_v)
    pltpu.sync_copy(p2_hbm.at[pl.ds(base, TPW)], idx2_v)
    pltpu.sync_copy(m1_hbm.at[pl.ds(base, TPW)], mv1_v)
    pltpu.sync_copy(m2_hbm.at[pl.ds(base, TPW)], mv2_v)
    pltpu.sync_copy(x_hbm.at[pl.ds(base, TPW)], rows_v)
    c1 = pltpu.async_copy(rows_v, xs_hbm.at[idx1_v], sem1)
    c2 = pltpu.async_copy(rows_v, xs_hbm.at[idx2_v], sem2)
    c3 = pltpu.async_copy(mv1_v, ws_hbm.at[idx1_v], sem1)
    c4 = pltpu.async_copy(mv2_v, ws_hbm.at[idx2_v], sem2)
    c1.wait()
    c2.wait()
    c3.wait()
    c4.wait()


def _dispatch(x, pos1, pos2, m1, m2):
    return pl.kernel(
        _dispatch_body,
        out_type=[
            jax.ShapeDtypeStruct((XS_ROWS, D), jnp.float32),
            jax.ShapeDtypeStruct((XS_ROWS,), jnp.float32),
        ],
        mesh=_sc_mesh(),
        scratch_types=[
            pltpu.VMEM((TPW,), jnp.int32),
            pltpu.VMEM((TPW,), jnp.int32),
            pltpu.VMEM((TPW,), jnp.float32),
            pltpu.VMEM((TPW,), jnp.float32),
            pltpu.VMEM((TPW, D), jnp.float32),
            pltpu.SemaphoreType.DMA,
            pltpu.SemaphoreType.DMA,
        ],
    )(x, pos1, pos2, m1, m2)


CCH = 16
NCH = TPW // CCH


def _combine_body(ys_hbm, p1_hbm, p2_hbm, out_hbm, idx1_v, idx2_v,
                  bufa1, bufa2, bufb1, bufb2, sema, semb):
    wid = lax.axis_index("s") * NC + lax.axis_index("c")
    base = wid * TPW
    pltpu.sync_copy(p1_hbm.at[wid], idx1_v)
    pltpu.sync_copy(p2_hbm.at[wid], idx2_v)

    def _adds(b1, b2):
        def _add_row(r, carry):
            for j in range(D // 16):
                sl = pl.ds(j * 16, 16)
                b1[r, sl] = b1[r, sl] + b2[r, sl]
            return carry
        lax.fori_loop(0, CCH, _add_row, 0)

    prev = None
    for s in range(NCH):
        b1, b2 = (bufa1, bufa2) if s % 2 == 0 else (bufb1, bufb2)
        sem = sema if s % 2 == 0 else semb
        c1 = pltpu.async_copy(ys_hbm.at[idx1_v.at[s]], b1, sem)
        c2 = pltpu.async_copy(ys_hbm.at[idx2_v.at[s]], b2, sem)
        if prev is not None:
            pc1, pc2, pb1, pb2, ps = prev
            pc1.wait()
            pc2.wait()
            _adds(pb1, pb2)
            pltpu.sync_copy(pb1, out_hbm.at[pl.ds(base + ps * CCH, CCH)])
        prev = (c1, c2, b1, b2, s)
    pc1, pc2, pb1, pb2, ps = prev
    pc1.wait()
    pc2.wait()
    _adds(pb1, pb2)
    pltpu.sync_copy(pb1, out_hbm.at[pl.ds(base + ps * CCH, CCH)])


def _combine(ys, pos1, pos2):
    return pl.kernel(
        _combine_body,
        out_type=jax.ShapeDtypeStruct((T, D), jnp.float32),
        mesh=_sc_mesh(),
        scratch_types=[
            pltpu.VMEM((NCH, CCH), jnp.int32),
            pltpu.VMEM((NCH, CCH), jnp.int32),
            pltpu.VMEM((CCH, D), jnp.float32),
            pltpu.VMEM((CCH, D), jnp.float32),
            pltpu.VMEM((CCH, D), jnp.float32),
            pltpu.VMEM((CCH, D), jnp.float32),
            pltpu.SemaphoreType.DMA,
            pltpu.SemaphoreType.DMA,
        ],
    )(ys, pos1.reshape(NW, NCH, CCH), pos2.reshape(NW, NCH, CCH))


def _cumsum0(v, n):
    k = 1
    while k < n:
        shifted = jnp.concatenate(
            [jnp.zeros((k,) + v.shape[1:], v.dtype), v[:-k]], axis=0)
        v = v + shifted
        k *= 2
    return v


def _cumsum1(v, n):
    k = 1
    while k < n:
        shifted = jnp.concatenate(
            [jnp.zeros(v.shape[:1] + (k,), v.dtype), v[:, :-k]], axis=1)
        v = v + shifted
        k *= 2
    return v


def _router_body(x_ref, gw_ref, m1_ref, m2_ref, p1_ref, p2_ref,
                 cnt_ref, off_ref):
    x = x_ref[...]
    logits = jnp.dot(x, gw_ref[...], preferred_element_type=jnp.float32)
    logits = jnp.tanh(logits / SOFTCAP) * SOFTCAP
    mx = jnp.max(logits, axis=1, keepdims=True)
    p = jnp.exp(logits - mx)
    probs = p / jnp.sum(p, axis=1, keepdims=True)
    cols = lax.broadcasted_iota(jnp.int32, (T, E), 1)
    m1 = jnp.max(probs, axis=1, keepdims=True)
    i1 = jnp.min(jnp.where(probs == m1, cols, E), axis=1, keepdims=True)
    p2 = jnp.where(cols == i1, -1.0, probs)
    m2 = jnp.max(p2, axis=1, keepdims=True)
    i2 = jnp.min(jnp.where(p2 == m2, cols, E), axis=1, keepdims=True)
    m1_ref[...] = m1
    m2_ref[...] = m2

    oh1 = cols == i1
    oh2 = cols == i2
    tot = oh1.astype(jnp.int32) + oh2.astype(jnp.int32)
    csum = _cumsum0(tot, T)
    cb = csum - tot
    counts = csum[T - 1:T, :]
    counts8 = (counts + 7) // 8 * 8
    off8 = _cumsum1(counts8, E) - counts8
    dest = cb + off8
    p1_ref[...] = jnp.sum(jnp.where(oh1, dest, 0), axis=1, keepdims=True)
    p2_ref[...] = jnp.sum(jnp.where(oh2, dest, 0), axis=1, keepdims=True)
    cnt_ref[...] = counts
    off_ref[...] = off8


def _router(x, gate_w):
    return pl.pallas_call(
        _router_body,
        in_specs=[
            pl.BlockSpec((T, D), lambda: (0, 0)),
            pl.BlockSpec((D, E), lambda: (0, 0)),
        ],
        out_specs=[
            pl.BlockSpec((T, 1), lambda: (0, 0)),
            pl.BlockSpec((T, 1), lambda: (0, 0)),
            pl.BlockSpec((T, 1), lambda: (0, 0)),
            pl.BlockSpec((T, 1), lambda: (0, 0)),
            pl.BlockSpec((1, E), lambda: (0, 0)),
            pl.BlockSpec((1, E), lambda: (0, 0)),
        ],
        out_shape=[
            jax.ShapeDtypeStruct((T, 1), jnp.float32),
            jax.ShapeDtypeStruct((T, 1), jnp.float32),
            jax.ShapeDtypeStruct((T, 1), jnp.int32),
            jax.ShapeDtypeStruct((T, 1), jnp.int32),
            jax.ShapeDtypeStruct((1, E), jnp.int32),
            jax.ShapeDtypeStruct((1, E), jnp.int32),
        ],
    )(x, gate_w)


def _gmm_body(off_ref, cnt_ref, xs_ref, ws_ref, w1_ref, w3_ref, w2_ref, ys_ref):
    e = pl.program_id(0)
    off = off_ref[e]
    cnt = cnt_ref[e]
    ntile = (cnt + TM - 1) // TM
    w1 = w1_ref[0]
    w3 = w3_ref[0]
    w2 = w2_ref[0]

    def body(i, carry):
        start = pl.multiple_of(off + i * TM, 8)
        xc = xs_ref[pl.ds(start, TM), :]
        g = jnp.dot(xc, w1, preferred_element_type=jnp.float32)
        u = jnp.dot(xc, w3, preferred_element_type=jnp.float32)
        h = jax.nn.gelu(g) * u
        y = jnp.dot(h, w2, preferred_element_type=jnp.float32)
        ys_ref[pl.ds(start, TM), :] = y * ws_ref[pl.ds(start, TM), :]
        return carry

    lax.fori_loop(0, ntile, body, 0)


def _gmm(off, cnt, xs, ws, w1, w3, w2):
    grid_spec = pltpu.PrefetchScalarGridSpec(
        num_scalar_prefetch=2,
        grid=(E,),
        in_specs=[
            pl.BlockSpec((XS_ROWS, D), lambda e, o, c: (0, 0)),
            pl.BlockSpec((XS_ROWS, 1), lambda e, o, c: (0, 0)),
            pl.BlockSpec((1, D, FF), lambda e, o, c: (e, 0, 0)),
            pl.BlockSpec((1, D, FF), lambda e, o, c: (e, 0, 0)),
            pl.BlockSpec((1, FF, D), lambda e, o, c: (e, 0, 0)),
        ],
        out_specs=pl.BlockSpec((XS_ROWS, D), lambda e, o, c: (0, 0)),
    )
    return pl.pallas_call(
        _gmm_body,
        grid_spec=grid_spec,
        out_shape=jax.ShapeDtypeStruct((XS_ROWS, D), jnp.float32),
        compiler_params=pltpu.CompilerParams(
            dimension_semantics=("arbitrary",),
        ),
    )(off, cnt, xs, ws, w1, w3, w2)


def kernel(hidden_states, gate_w, w1, w3, w2):
    x = hidden_states
    m1, m2, pos1, pos2, counts, off8 = _router(x, gate_w)
    m1 = m1[:, 0]
    m2 = m2[:, 0]
    pos1 = pos1[:, 0]
    pos2 = pos2[:, 0]
    counts = counts[0]
    off8 = off8[0]

    xs, ws = _dispatch(x, pos1, pos2, m1, m2)

    ys = _gmm(off8, counts, xs, ws[:, None], w1, w3, w2)

    out = _combine(ys, pos1, pos2)
    return out


# speedup vs baseline: 1.1336x; 1.1336x over previous
import functools

import jax
import jax.numpy as jnp
from jax import lax
from jax.experimental import pallas as pl
from jax.experimental.pallas import tpu as pltpu
from jax.experimental.pallas import tpu_sc as plsc

T, D, FF, E, K = 2048, 1024, 512, 64, 2
SOFTCAP = 30.0
A = T * K
TM = 128
XS_ROWS = 4736

NC, NS = 2, 16
NW = NC * NS
TPW = T // NW
CPW = TPW // 2

def _sc_mesh():
    return plsc.VectorSubcoreMesh(core_axis_name="c", subcore_axis_name="s",
                                  num_cores=NC, num_subcores=NS)


def _dispatch_body(x_hbm, p1_hbm, p2_hbm, xs_hbm,
                   idx1_v, idx2_v, rows_v, sem1, sem2):
    wid = lax.axis_index("s") * NC + lax.axis_index("c")
    base = wid * TPW
    pltpu.sync_copy(p1_hbm.at[pl.ds(base, TPW)], idx1_v)
    pltpu.sync_copy(p2_hbm.at[pl.ds(base, TPW)], idx2_v)
    pltpu.sync_copy(x_hbm.at[pl.ds(base, TPW)], rows_v)
    c1 = pltpu.async_copy(rows_v, xs_hbm.at[idx1_v], sem1)
    c2 = pltpu.async_copy(rows_v, xs_hbm.at[idx2_v], sem2)
    c1.wait()
    c2.wait()


def _dispatch(x, pos1, pos2):
    return pl.kernel(
        _dispatch_body,
        out_type=jax.ShapeDtypeStruct((XS_ROWS, D), jnp.float32),
        mesh=_sc_mesh(),
        scratch_types=[
            pltpu.VMEM((TPW,), jnp.int32),
            pltpu.VMEM((TPW,), jnp.int32),
            pltpu.VMEM((TPW, D), jnp.float32),
            pltpu.SemaphoreType.DMA,
            pltpu.SemaphoreType.DMA,
        ],
    )(x, pos1, pos2)


CCH = 16
NCH = TPW // CCH


def _combine_body(ys_hbm, p1_hbm, p2_hbm, out_hbm, idx1_v, idx2_v,
                  bufa1, bufa2, bufb1, bufb2, sema, semb):
    wid = lax.axis_index("s") * NC + lax.axis_index("c")
    base = wid * TPW
    pltpu.sync_copy(p1_hbm.at[wid], idx1_v)
    pltpu.sync_copy(p2_hbm.at[wid], idx2_v)

    def _adds(b1, b2):
        def _add_row(r, carry):
            for j in range(D // 16):
                sl = pl.ds(j * 16, 16)
                b1[r, sl] = b1[r, sl] + b2[r, sl]
            return carry
        lax.fori_loop(0, CCH, _add_row, 0)

    prev = None
    for s in range(NCH):
        b1, b2 = (bufa1, bufa2) if s % 2 == 0 else (bufb1, bufb2)
        sem = sema if s % 2 == 0 else semb
        c1 = pltpu.async_copy(ys_hbm.at[idx1_v.at[s]], b1, sem)
        c2 = pltpu.async_copy(ys_hbm.at[idx2_v.at[s]], b2, sem)
        if prev is not None:
            pc1, pc2, pb1, pb2, ps = prev
            pc1.wait()
            pc2.wait()
            _adds(pb1, pb2)
            pltpu.sync_copy(pb1, out_hbm.at[pl.ds(base + ps * CCH, CCH)])
        prev = (c1, c2, b1, b2, s)
    pc1, pc2, pb1, pb2, ps = prev
    pc1.wait()
    pc2.wait()
    _adds(pb1, pb2)
    pltpu.sync_copy(pb1, out_hbm.at[pl.ds(base + ps * CCH, CCH)])


def _combine(ys, pos1, pos2):
    return pl.kernel(
        _combine_body,
        out_type=jax.ShapeDtypeStruct((T, D), jnp.float32),
        mesh=_sc_mesh(),
        scratch_types=[
            pltpu.VMEM((NCH, CCH), jnp.int32),
            pltpu.VMEM((NCH, CCH), jnp.int32),
            pltpu.VMEM((CCH, D), jnp.float32),
            pltpu.VMEM((CCH, D), jnp.float32),
            pltpu.VMEM((CCH, D), jnp.float32),
            pltpu.VMEM((CCH, D), jnp.float32),
            pltpu.SemaphoreType.DMA,
            pltpu.SemaphoreType.DMA,
        ],
    )(ys, pos1.reshape(NW, NCH, CCH), pos2.reshape(NW, NCH, CCH))


def _cumsum0(v, n):
    k = 1
    while k < n:
        shifted = jnp.concatenate(
            [jnp.zeros((k,) + v.shape[1:], v.dtype), v[:-k]], axis=0)
        v = v + shifted
        k *= 2
    return v


def _cumsum1(v, n):
    k = 1
    while k < n:
        shifted = jnp.concatenate(
            [jnp.zeros(v.shape[:1] + (k,), v.dtype), v[:, :-k]], axis=1)
        v = v + shifted
        k *= 2
    return v


def _router_body(x_ref, gw_ref, m1_ref, m2_ref, p1_ref, p2_ref,
                 cnt_ref, off_ref):
    x = x_ref[...]
    logits = jnp.dot(x, gw_ref[...], preferred_element_type=jnp.float32)
    logits = jnp.tanh(logits / SOFTCAP) * SOFTCAP
    mx = jnp.max(logits, axis=1, keepdims=True)
    p = jnp.exp(logits - mx)
    probs = p / jnp.sum(p, axis=1, keepdims=True)
    cols = lax.broadcasted_iota(jnp.int32, (T, E), 1)
    m1 = jnp.max(probs, axis=1, keepdims=True)
    i1 = jnp.min(jnp.where(probs == m1, cols, E), axis=1, keepdims=True)
    p2 = jnp.where(cols == i1, -1.0, probs)
    m2 = jnp.max(p2, axis=1, keepdims=True)
    i2 = jnp.min(jnp.where(p2 == m2, cols, E), axis=1, keepdims=True)
    m1_ref[...] = m1
    m2_ref[...] = m2

    oh1 = cols == i1
    oh2 = cols == i2
    tot = oh1.astype(jnp.int32) + oh2.astype(jnp.int32)
    csum = _cumsum0(tot, T)
    cb = csum - tot
    counts = csum[T - 1:T, :]
    counts8 = (counts + 7) // 8 * 8
    off8 = _cumsum1(counts8, E) - counts8
    dest = cb + off8
    p1_ref[...] = jnp.sum(jnp.where(oh1, dest, 0), axis=1, keepdims=True)
    p2_ref[...] = jnp.sum(jnp.where(oh2, dest, 0), axis=1, keepdims=True)
    cnt_ref[...] = counts
    off_ref[...] = off8


def _router(x, gate_w):
    return pl.pallas_call(
        _router_body,
        in_specs=[
            pl.BlockSpec((T, D), lambda: (0, 0)),
            pl.BlockSpec((D, E), lambda: (0, 0)),
        ],
        out_specs=[
            pl.BlockSpec((T, 1), lambda: (0, 0)),
            pl.BlockSpec((T, 1), lambda: (0, 0)),
            pl.BlockSpec((T, 1), lambda: (0, 0)),
            pl.BlockSpec((T, 1), lambda: (0, 0)),
            pl.BlockSpec((1, E), lambda: (0, 0)),
            pl.BlockSpec((1, E), lambda: (0, 0)),
        ],
        out_shape=[
            jax.ShapeDtypeStruct((T, 1), jnp.float32),
            jax.ShapeDtypeStruct((T, 1), jnp.float32),
            jax.ShapeDtypeStruct((T, 1), jnp.int32),
            jax.ShapeDtypeStruct((T, 1), jnp.int32),
            jax.ShapeDtypeStruct((1, E), jnp.int32),
            jax.ShapeDtypeStruct((1, E), jnp.int32),
        ],
    )(x, gate_w)


def _gmm_body(off_ref, cnt_ref, xs_ref, ws_ref, w1_ref, w3_ref, w2_ref, ys_ref):
    e = pl.program_id(0)
    off = off_ref[e]
    cnt = cnt_ref[e]
    ntile = (cnt + TM - 1) // TM
    w1 = w1_ref[0]
    w3 = w3_ref[0]
    w2 = w2_ref[0]

    def body(i, carry):
        start = pl.multiple_of(off + i * TM, 8)
        xc = xs_ref[pl.ds(start, TM), :]
        g = jnp.dot(xc, w1, preferred_element_type=jnp.float32)
        u = jnp.dot(xc, w3, preferred_element_type=jnp.float32)
        h = jax.nn.gelu(g) * u
        y = jnp.dot(h, w2, preferred_element_type=jnp.float32)
        ys_ref[pl.ds(start, TM), :] = y * ws_ref[pl.ds(start, TM), :]
        return carry

    lax.fori_loop(0, ntile, body, 0)


def _gmm(off, cnt, xs, ws, w1, w3, w2):
    grid_spec = pltpu.PrefetchScalarGridSpec(
        num_scalar_prefetch=2,
        grid=(E,),
        in_specs=[
            pl.BlockSpec((XS_ROWS, D), lambda e, o, c: (0, 0)),
            pl.BlockSpec((XS_ROWS, 1), lambda e, o, c: (0, 0)),
            pl.BlockSpec((1, D, FF), lambda e, o, c: (e, 0, 0)),
            pl.BlockSpec((1, D, FF), lambda e, o, c: (e, 0, 0)),
            pl.BlockSpec((1, FF, D), lambda e, o, c: (e, 0, 0)),
        ],
        out_specs=pl.BlockSpec((XS_ROWS, D), lambda e, o, c: (0, 0)),
    )
    return pl.pallas_call(
        _gmm_body,
        grid_spec=grid_spec,
        out_shape=jax.ShapeDtypeStruct((XS_ROWS, D), jnp.float32),
        compiler_params=pltpu.CompilerParams(
            dimension_semantics=("arbitrary",),
        ),
    )(off, cnt, xs, ws, w1, w3, w2)


def kernel(hidden_states, gate_w, w1, w3, w2):
    x = hidden_states
    m1, m2, pos1, pos2, counts, off8 = _router(x, gate_w)
    m1 = m1[:, 0]
    m2 = m2[:, 0]
    pos1 = pos1[:, 0]
    pos2 = pos2[:, 0]
    counts = counts[0]
    off8 = off8[0]

    ws = (jnp.zeros((XS_ROWS,), jnp.float32)
          .at[pos1].set(m1).at[pos2].set(m2))

    xs = _dispatch(x, pos1, pos2)

    ys = _gmm(off8, counts, xs, ws[:, None], w1, w3, w2)

    out = _combine(ys, pos1, pos2)
    return out
